# baseline (speedup 1.0000x reference)
import dataclasses
import functools

import jax
import jax.numpy as jnp
from jax import lax
from jax.experimental import pallas as pl
from jax.experimental.pallas import tpu as pltpu
from jax.experimental.pallas import tpu_sc as plsc

B, N, K = 256, 65536, 10
NH = N // 2
NB = 8192
GRID_H = NH // NB

NUM_CORES = 2
NUM_SUBCORES = 16
NUM_WORKERS = NUM_CORES * NUM_SUBCORES
CH = NH // NUM_WORKERS
LANES = 16


def _sc_prep_body(rates_t_hbm, aux_hbm, r_v, w_v, idx_v, sem):
    wid = lax.axis_index("s") * NUM_CORES + lax.axis_index("c")
    base = wid * CH
    pltpu.async_copy(rates_t_hbm.at[:, pl.ds(NH + base, CH)], r_v, sem).wait()

    @plsc.parallel_loop(0, CH, step=LANES, unroll=2)
    def _(i16):
        v = r_v[0, pl.ds(i16, LANES)]
        l1 = jnp.abs(v)
        m = v
        idx = jnp.zeros((LANES,), jnp.float32)
        for k in range(1, K):
            vk = r_v[k, pl.ds(i16, LANES)]
            l1 = l1 + jnp.abs(vk)
            gt = vk > m
            m = jnp.where(gt, vk, m)
            idx = jnp.where(gt, float(k), idx)
        w = m / jnp.maximum(l1, 1e-12)
        w_v[pl.ds(i16, LANES)] = w
        idx_v[pl.ds(i16, LANES)] = idx

    pltpu.async_copy(w_v, aux_hbm.at[0, pl.ds(base, CH)], sem).wait()
    pltpu.async_copy(idx_v, aux_hbm.at[1, pl.ds(base, CH)], sem).wait()


def _sc_prep(rates_t):
    mesh = plsc.VectorSubcoreMesh(core_axis_name="c", subcore_axis_name="s")
    return pl.kernel(
        _sc_prep_body,
        out_type=jax.ShapeDtypeStruct((2, NH), jnp.float32),
        mesh=mesh,
        scratch_types=[
            pltpu.VMEM((K, CH), jnp.float32),
            pltpu.VMEM((CH,), jnp.float32),
            pltpu.VMEM((CH,), jnp.float32),
            pltpu.SemaphoreType.DMA,
        ],
    )(rates_t)


def _tc_a_kernel(inputs_ref, rates_t_ref, acc_ref, occ_ref):
    i = pl.program_id(0)

    rt = rates_t_ref[...]
    l1 = jnp.maximum(jnp.sum(jnp.abs(rt), axis=0, keepdims=True), 1e-12)
    m = jnp.max(rt, axis=0, keepdims=True)
    sub = lax.broadcasted_iota(jnp.int32, rt.shape, 0)
    idx = jnp.min(jnp.where(rt == m, sub, K), axis=0, keepdims=True)
    onehot = sub == idx
    s_t = jnp.where(onehot, m / l1, 0.0)

    part = lax.dot_general(
        inputs_ref[...], s_t,
        dimension_numbers=(((1,), (1,)), ((), ())),
        preferred_element_type=jnp.float32,
    )
    occ_part = jnp.sum(onehot.astype(jnp.float32), axis=1, keepdims=True)

    @pl.when(i == 0)
    def _():
        acc_ref[...] = jnp.zeros_like(acc_ref)
        occ_ref[...] = jnp.zeros_like(occ_ref)

    acc_ref[...] += part
    occ_ref[...] += occ_part


def _tc_b_kernel(inputs_ref, aux_ref, acc_in_ref, occ_in_ref, out_ref,
                 acc_ref, occ_ref):
    i = pl.program_id(0)

    wrow = aux_ref[0:1, :]
    irow = aux_ref[1:2, :]
    sub = lax.broadcasted_iota(jnp.int32, (K, NB), 0).astype(jnp.float32)
    onehot = sub == irow
    s_t = jnp.where(onehot, wrow, 0.0)

    part = lax.dot_general(
        inputs_ref[...], s_t,
        dimension_numbers=(((1,), (1,)), ((), ())),
        preferred_element_type=jnp.float32,
    )
    occ_part = jnp.sum(onehot.astype(jnp.float32), axis=1, keepdims=True)

    @pl.when(i == 0)
    def _():
        acc_ref[...] = acc_in_ref[...]
        occ_ref[...] = occ_in_ref[...]

    acc_ref[...] += part
    occ_ref[...] += occ_part

    @pl.when(i == GRID_H - 1)
    def _():
        occ = occ_ref[...].reshape(1, K)
        q = acc_ref[...] / occ
        q = jnp.where(jnp.isnan(q), 0.0, q)
        q = jnp.where(q == jnp.inf, 0.0, q)
        q = jnp.where(q == -jnp.inf, jnp.finfo(jnp.float32).min, q)
        out_ref[...] = q


@jax.jit
def kernel(inputs, rates):
    rates_t = rates.T
    aux = _sc_prep(rates_t)

    acc_a, occ_a = pl.pallas_call(
        _tc_a_kernel,
        grid=(GRID_H,),
        in_specs=[
            pl.BlockSpec((B, NB), lambda i: (0, i)),
            pl.BlockSpec((K, NB), lambda i: (0, i)),
        ],
        out_specs=[
            pl.BlockSpec((B, K), lambda i: (0, 0)),
            pl.BlockSpec((K, 1), lambda i: (0, 0)),
        ],
        out_shape=[
            jax.ShapeDtypeStruct((B, K), jnp.float32),
            jax.ShapeDtypeStruct((K, 1), jnp.float32),
        ],
        compiler_params=pltpu.CompilerParams(
            dimension_semantics=("arbitrary",),
        ),
    )(inputs, rates_t)

    return pl.pallas_call(
        _tc_b_kernel,
        grid=(GRID_H,),
        in_specs=[
            pl.BlockSpec((B, NB), lambda i: (0, i + GRID_H)),
            pl.BlockSpec((2, NB), lambda i: (0, i)),
            pl.BlockSpec((B, K), lambda i: (0, 0)),
            pl.BlockSpec((K, 1), lambda i: (0, 0)),
        ],
        out_specs=pl.BlockSpec((B, K), lambda i: (0, 0)),
        out_shape=jax.ShapeDtypeStruct((B, K), jnp.float32),
        scratch_shapes=[
            pltpu.VMEM((B, K), jnp.float32),
            pltpu.VMEM((K, 1), jnp.float32),
        ],
        compiler_params=pltpu.CompilerParams(
            dimension_semantics=("arbitrary",),
        ),
    )(inputs, aux, acc_a, occ_a)


# speedup vs baseline: 1.0022x; 1.0022x over previous
import functools

import jax
import jax.numpy as jnp
from jax import lax
from jax.experimental import pallas as pl
from jax.experimental.pallas import tpu as pltpu
from jax.experimental.pallas import tpu_sc as plsc

B, N, K = 256, 65536, 10
NB = 8192
GRID = N // NB

NUM_CORES = 2
NUM_SUBCORES = 16
NUM_WORKERS = NUM_CORES * NUM_SUBCORES
CH = N // NUM_WORKERS
LANES = 16


def _sc_count_body(rates_t_hbm, cnt_hbm, r_v, cnt_v, sem):
    wid = lax.axis_index("s") * NUM_CORES + lax.axis_index("c")
    base = wid * CH
    pltpu.async_copy(rates_t_hbm.at[:, pl.ds(base, CH)], r_v, sem).wait()

    init = tuple(jnp.zeros((LANES,), jnp.float32) for _ in range(K))

    def body(i16, counts):
        v = r_v[0, pl.ds(i16, LANES)]
        m = v
        idx = jnp.zeros((LANES,), jnp.int32)
        for k in range(1, K):
            vk = r_v[k, pl.ds(i16, LANES)]
            gt = vk > m
            m = jnp.where(gt, vk, m)
            idx = jnp.where(gt, k, idx)
        return tuple(
            counts[k] + jnp.where(idx == k, 1.0, 0.0) for k in range(K)
        )

    counts = plsc.parallel_loop(0, CH, step=LANES, unroll=2, carry=init)(body)
    for k in range(K):
        cnt_v[k, :] = counts[k]
    pltpu.async_copy(cnt_v, cnt_hbm.at[wid], sem).wait()


def _sc_counts(rates_t):
    mesh = plsc.VectorSubcoreMesh(core_axis_name="c", subcore_axis_name="s")
    return pl.kernel(
        _sc_count_body,
        out_type=jax.ShapeDtypeStruct((NUM_WORKERS, K, LANES), jnp.float32),
        mesh=mesh,
        scratch_types=[
            pltpu.VMEM((K, CH), jnp.float32),
            pltpu.VMEM((K, LANES), jnp.float32),
            pltpu.SemaphoreType.DMA,
        ],
    )(rates_t)


def _tc_matmul_kernel(inputs_ref, rates_t_ref, acc_out_ref, acc_ref):
    i = pl.program_id(0)

    rt = rates_t_ref[...]
    l1 = jnp.maximum(jnp.sum(jnp.abs(rt), axis=0, keepdims=True), 1e-12)
    m = jnp.max(rt, axis=0, keepdims=True)
    sub = lax.broadcasted_iota(jnp.int32, rt.shape, 0)
    idx = jnp.min(jnp.where(rt == m, sub, K), axis=0, keepdims=True)
    s_t = jnp.where(sub == idx, m / l1, 0.0)

    part = lax.dot_general(
        inputs_ref[...], s_t,
        dimension_numbers=(((1,), (1,)), ((), ())),
        preferred_element_type=jnp.float32,
    )

    @pl.when(i == 0)
    def _():
        acc_ref[...] = jnp.zeros_like(acc_ref)

    acc_ref[...] += part

    @pl.when(i == GRID - 1)
    def _():
        acc_out_ref[...] = acc_ref[...]


def _tc_final_kernel(acc_ref, cnt_ref, out_ref):
    occ = jnp.sum(jnp.sum(cnt_ref[...], axis=2), axis=0, keepdims=True)
    q = acc_ref[...] / occ
    q = jnp.where(jnp.isnan(q), 0.0, q)
    q = jnp.where(q == jnp.inf, 0.0, q)
    q = jnp.where(q == -jnp.inf, jnp.finfo(jnp.float32).min, q)
    out_ref[...] = q


@jax.jit
def kernel(inputs, rates):
    rates_t = rates.T
    cnt = _sc_counts(rates_t)

    acc = pl.pallas_call(
        _tc_matmul_kernel,
        grid=(GRID,),
        in_specs=[
            pl.BlockSpec((B, NB), lambda i: (0, i)),
            pl.BlockSpec((K, NB), lambda i: (0, i)),
        ],
        out_specs=pl.BlockSpec((B, K), lambda i: (0, 0)),
        out_shape=jax.ShapeDtypeStruct((B, K), jnp.float32),
        scratch_shapes=[pltpu.VMEM((B, K), jnp.float32)],
        compiler_params=pltpu.CompilerParams(
            dimension_semantics=("arbitrary",),
        ),
    )(inputs, rates_t)

    return pl.pallas_call(
        _tc_final_kernel,
        in_specs=[
            pl.BlockSpec((B, K), lambda: (0, 0)),
            pl.BlockSpec((NUM_WORKERS, K, LANES), lambda: (0, 0, 0)),
        ],
        out_specs=pl.BlockSpec((B, K), lambda: (0, 0)),
        out_shape=jax.ShapeDtypeStruct((B, K), jnp.float32),
    )(acc, cnt)
